# baseline (speedup 1.0000x reference)
import functools
import numpy as np
import jax
import jax.numpy as jnp
from jax.experimental import pallas as pl
from jax.experimental.pallas import tpu as pltpu

K = 8
_ROT_A = (13, 15, 26, 6)
_ROT_B = (17, 29, 16, 24)
_KS0 = np.uint32(0)
_KS1 = np.uint32(42)
_KS2 = np.uint32(0x1BD11BDA) ^ _KS0 ^ _KS1
_TINY = np.float32(np.finfo(np.float32).tiny)
_CH = 512
_IMAX = np.int32(2**31 - 1)


def _rotl(v, r):
    return (v << jnp.uint32(r)) | (v >> jnp.uint32(32 - r))


def _threefry_bits(cnt):
    x0 = jnp.zeros_like(cnt) + _KS0
    x1 = cnt + _KS1
    inject = ((_KS1, _KS2), (_KS2, _KS0), (_KS0, _KS1), (_KS1, _KS2),
              (_KS2, _KS0))
    for g in range(5):
        for r in (_ROT_A if g % 2 == 0 else _ROT_B):
            x0 = x0 + x1
            x1 = _rotl(x1, r)
            x1 = x1 ^ x0
        a, b = inject[g]
        x0 = x0 + a
        x1 = x1 + (b + np.uint32(g + 1))
    return x0 ^ x1


def _gumbel_from_bits(bits):
    fb = (bits >> jnp.uint32(9)) | jnp.uint32(0x3F800000)
    f = jax.lax.bitcast_convert_type(fb, jnp.float32) - jnp.float32(1.0)
    u = jnp.maximum(_TINY, f + _TINY)
    return -jnp.log(-jnp.log(u))


def _sampler_kernel(x_ref, out_ref, *, vocab, n_full, tail_w, tail_n):
    r = pl.program_id(0)
    base_flat = jnp.uint32(r) * jnp.uint32(K * vocab)
    k_off = jax.lax.broadcasted_iota(jnp.uint32, (K, _CH), 0) * jnp.uint32(vocab)

    def score_chunk(base_j, width):
        lane = jax.lax.broadcasted_iota(jnp.uint32, (K, width), 1)
        cnt = (base_flat + k_off[:, :width] + lane
               + jnp.asarray(base_j).astype(jnp.uint32))
        g = _gumbel_from_bits(_threefry_bits(cnt))
        xv = x_ref[0, :, pl.ds(base_j, width)]
        logits = jnp.log(jnp.maximum(xv, jnp.float32(1e-30)))
        return g + logits

    def body(c, carry):
        best_v, best_j = carry
        base_j = c * _CH
        s = score_chunk(base_j, _CH)
        jidx = base_j + jax.lax.broadcasted_iota(jnp.int32, (K, _CH), 1)
        upd = s > best_v
        return (jnp.where(upd, s, best_v), jnp.where(upd, jidx, best_j))

    best_v = jnp.full((K, _CH), -jnp.inf, jnp.float32)
    best_j = jnp.zeros((K, _CH), jnp.int32)
    best_v, best_j = jax.lax.fori_loop(0, n_full, body, (best_v, best_j))

    m = jnp.max(best_v, axis=1, keepdims=True)
    cand = jnp.where(best_v == m, best_j, _IMAX)
    idx = jnp.min(cand, axis=1, keepdims=True)

    if tail_n > 0:
        base_j = vocab - tail_w
        s = score_chunk(base_j, tail_w)
        lane_i = jax.lax.broadcasted_iota(jnp.int32, (K, tail_w), 1)
        jidx = base_j + lane_i
        mt = jnp.max(s, axis=1, keepdims=True)
        ct = jnp.where(s == mt, jidx, _IMAX)
        it = jnp.min(ct, axis=1, keepdims=True)
        take = mt > m
        idx = jnp.where(take, it, idx)

    out_ref[0, :, :] = jnp.broadcast_to(idx, (K, 128))


@jax.jit
def kernel(x):
    rows, vocab = x.shape
    n_full = vocab // _CH
    tail = vocab - n_full * _CH
    tail_w = ((tail + 127) // 128) * 128
    body = functools.partial(_sampler_kernel, vocab=vocab, n_full=n_full,
                             tail_w=tail_w, tail_n=tail)
    out = pl.pallas_call(
        body,
        grid=(rows,),
        in_specs=[pl.BlockSpec((1, 1, vocab), lambda i: (i, 0, 0))],
        out_specs=pl.BlockSpec((1, K, 128), lambda i: (i, 0, 0)),
        out_shape=jax.ShapeDtypeStruct((rows, K, 128), jnp.int32),
        compiler_params=pltpu.CompilerParams(
            dimension_semantics=("parallel",)),
    )(x[:, None, :])
    return out[:, :, 0]


# speedup vs baseline: 1.4389x; 1.4389x over previous
import functools
import numpy as np
import jax
import jax.numpy as jnp
from jax.experimental import pallas as pl
from jax.experimental.pallas import tpu as pltpu

K = 8
_ROT_A = (13, 15, 26, 6)
_ROT_B = (17, 29, 16, 24)
_KS0 = np.uint32(0)
_KS1 = np.uint32(42)
_KS2 = np.uint32(0x1BD11BDA) ^ _KS0 ^ _KS1
_TINY = np.float32(np.finfo(np.float32).tiny)
_CH = 1024
_IMAX = np.int32(2**31 - 1)


def _rotl(v, r):
    return (v << jnp.uint32(r)) | (v >> jnp.uint32(32 - r))


def _threefry_bits(x1):
    x0 = x1
    x1 = _rotl(x1, 13) ^ x0
    for r in _ROT_A[1:]:
        x0 = x0 + x1
        x1 = _rotl(x1, r) ^ x0
    x0 = x0 + _KS1
    x1 = x1 + (_KS2 + np.uint32(1))
    inject = ((_KS2, np.uint32(2)), (None, _KS1 + np.uint32(3)),
              (_KS1, _KS2 + np.uint32(4)), (_KS2, np.uint32(5)))
    for g in range(4):
        for r in (_ROT_B if g % 2 == 0 else _ROT_A):
            x0 = x0 + x1
            x1 = _rotl(x1, r) ^ x0
        a, b = inject[g]
        if a is not None:
            x0 = x0 + a
        x1 = x1 + b
    return x0 ^ x1


def _gumbel_from_bits(bits):
    fb = (bits >> jnp.uint32(9)) | jnp.uint32(0x3F800000)
    f = jax.lax.bitcast_convert_type(fb, jnp.float32) - jnp.float32(1.0)
    u = jnp.maximum(_TINY, f + _TINY)
    return -jnp.log(-jnp.log(u))


def _sampler_kernel(x_ref, out_ref, *, vocab, n_full, tail_w, tail_n):
    r = pl.program_id(0)
    base_flat = jnp.uint32(r) * jnp.uint32(K * vocab) + _KS1
    cnt0 = (base_flat
            + jax.lax.broadcasted_iota(jnp.uint32, (K, _CH), 0)
            * jnp.uint32(vocab)
            + jax.lax.broadcasted_iota(jnp.uint32, (K, _CH), 1))

    def score_chunk(base_j, width):
        cnt = cnt0[:, :width] + jnp.asarray(base_j).astype(jnp.uint32)
        g = _gumbel_from_bits(_threefry_bits(cnt))
        xv = x_ref[0, :, pl.ds(base_j, width)]
        logits = jnp.log(jnp.maximum(xv, jnp.float32(1e-30)))
        return g + logits

    def body(c, carry):
        best_v, best_j = carry
        base_j = c * _CH
        s = score_chunk(base_j, _CH)
        jidx = base_j + jax.lax.broadcasted_iota(jnp.int32, (K, _CH), 1)
        upd = s > best_v
        return (jnp.where(upd, s, best_v), jnp.where(upd, jidx, best_j))

    best_v = jnp.full((K, _CH), -jnp.inf, jnp.float32)
    best_j = jnp.zeros((K, _CH), jnp.int32)
    best_v, best_j = jax.lax.fori_loop(0, n_full, body, (best_v, best_j))

    m = jnp.max(best_v, axis=1, keepdims=True)
    cand = jnp.where(best_v == m, best_j, _IMAX)
    idx = jnp.min(cand, axis=1, keepdims=True)

    if tail_n > 0:
        base_j = vocab - tail_w
        s = score_chunk(base_j, tail_w)
        lane_i = jax.lax.broadcasted_iota(jnp.int32, (K, tail_w), 1)
        jidx = base_j + lane_i
        mt = jnp.max(s, axis=1, keepdims=True)
        ct = jnp.where(s == mt, jidx, _IMAX)
        it = jnp.min(ct, axis=1, keepdims=True)
        take = mt > m
        idx = jnp.where(take, it, idx)

    out_ref[0, :, :] = jnp.broadcast_to(idx, (K, 128))


@jax.jit
def kernel(x):
    rows, vocab = x.shape
    n_full = vocab // _CH
    tail = vocab - n_full * _CH
    tail_w = ((tail + 127) // 128) * 128
    body = functools.partial(_sampler_kernel, vocab=vocab, n_full=n_full,
                             tail_w=tail_w, tail_n=tail)
    out = pl.pallas_call(
        body,
        grid=(rows,),
        in_specs=[pl.BlockSpec((1, 1, vocab), lambda i: (i, 0, 0))],
        out_specs=pl.BlockSpec((1, K, 128), lambda i: (i, 0, 0)),
        out_shape=jax.ShapeDtypeStruct((rows, K, 128), jnp.int32),
        compiler_params=pltpu.CompilerParams(
            dimension_semantics=("parallel",)),
    )(x[:, None, :])
    return out[:, :, 0]


# speedup vs baseline: 1.4853x; 1.0323x over previous
import functools
import numpy as np
import jax
import jax.numpy as jnp
from jax.experimental import pallas as pl
from jax.experimental.pallas import tpu as pltpu

K = 8
_ROT_A = (13, 15, 26, 6)
_ROT_B = (17, 29, 16, 24)
_KS0 = np.uint32(0)
_KS1 = np.uint32(42)
_KS2 = np.uint32(0x1BD11BDA) ^ _KS0 ^ _KS1
_TINY = np.float32(np.finfo(np.float32).tiny)
_CH = 2048
_IMAX = np.int32(2**31 - 1)


def _rotl(v, r):
    return (v << jnp.uint32(r)) | (v >> jnp.uint32(32 - r))


def _threefry_bits(x1):
    x0 = x1
    x1 = _rotl(x1, 13) ^ x0
    for r in _ROT_A[1:]:
        x0 = x0 + x1
        x1 = _rotl(x1, r) ^ x0
    x0 = x0 + _KS1
    x1 = x1 + (_KS2 + np.uint32(1))
    inject = ((_KS2, np.uint32(2)), (None, _KS1 + np.uint32(3)),
              (_KS1, _KS2 + np.uint32(4)), (_KS2, np.uint32(5)))
    for g in range(4):
        for r in (_ROT_B if g % 2 == 0 else _ROT_A):
            x0 = x0 + x1
            x1 = _rotl(x1, r) ^ x0
        a, b = inject[g]
        if a is not None:
            x0 = x0 + a
        x1 = x1 + b
    return x0 ^ x1


def _gumbel_from_bits(bits):
    fb = (bits >> jnp.uint32(9)) | jnp.uint32(0x3F800000)
    f = jax.lax.bitcast_convert_type(fb, jnp.float32) - jnp.float32(1.0)
    u = jnp.maximum(_TINY, f + _TINY)
    return -jnp.log(-jnp.log(u))


def _sampler_kernel(x_ref, out_ref, *, vocab, n_full, tail_w, tail_n):
    r = pl.program_id(0)
    base_flat = jnp.uint32(r) * jnp.uint32(K * vocab) + _KS1
    cnt0 = (base_flat
            + jax.lax.broadcasted_iota(jnp.uint32, (K, _CH), 0)
            * jnp.uint32(vocab)
            + jax.lax.broadcasted_iota(jnp.uint32, (K, _CH), 1))

    def score_chunk(base_j, width):
        cnt = cnt0[:, :width] + jnp.asarray(base_j).astype(jnp.uint32)
        g = _gumbel_from_bits(_threefry_bits(cnt))
        xv = x_ref[0, :, pl.ds(base_j, width)]
        logits = jnp.log(jnp.maximum(xv, jnp.float32(1e-30)))
        return g + logits

    def body(c, carry):
        best_v, best_j = carry
        base_j = c * _CH
        s = score_chunk(base_j, _CH)
        jidx = base_j + jax.lax.broadcasted_iota(jnp.int32, (K, _CH), 1)
        upd = s > best_v
        return (jnp.where(upd, s, best_v), jnp.where(upd, jidx, best_j))

    best_v = jnp.full((K, _CH), -jnp.inf, jnp.float32)
    best_j = jnp.zeros((K, _CH), jnp.int32)
    best_v, best_j = jax.lax.fori_loop(0, n_full, body, (best_v, best_j))

    m = jnp.max(best_v, axis=1, keepdims=True)
    cand = jnp.where(best_v == m, best_j, _IMAX)
    idx = jnp.min(cand, axis=1, keepdims=True)

    if tail_n > 0:
        base_j = vocab - tail_w
        s = score_chunk(base_j, tail_w)
        lane_i = jax.lax.broadcasted_iota(jnp.int32, (K, tail_w), 1)
        jidx = base_j + lane_i
        mt = jnp.max(s, axis=1, keepdims=True)
        ct = jnp.where(s == mt, jidx, _IMAX)
        it = jnp.min(ct, axis=1, keepdims=True)
        take = mt > m
        idx = jnp.where(take, it, idx)

    out_ref[0, :, :] = jnp.broadcast_to(idx, (K, 128))


@jax.jit
def kernel(x):
    rows, vocab = x.shape
    n_full = vocab // _CH
    tail = vocab - n_full * _CH
    tail_w = ((tail + 127) // 128) * 128
    body = functools.partial(_sampler_kernel, vocab=vocab, n_full=n_full,
                             tail_w=tail_w, tail_n=tail)
    out = pl.pallas_call(
        body,
        grid=(rows,),
        in_specs=[pl.BlockSpec((1, 1, vocab), lambda i: (i, 0, 0))],
        out_specs=pl.BlockSpec((1, K, 128), lambda i: (i, 0, 0)),
        out_shape=jax.ShapeDtypeStruct((rows, K, 128), jnp.int32),
        compiler_params=pltpu.CompilerParams(
            dimension_semantics=("parallel",)),
    )(x[:, None, :])
    return out[:, :, 0]
